# Initial kernel scaffold; baseline (speedup 1.0000x reference)
import jax
import jax.numpy as jnp
from jax.experimental import pallas as pl


def kernel(x, edge_index, W, b):
    raise NotImplementedError("write your pallas kernel here")


# speedup vs baseline: 6.2871x; 6.2871x over previous
import functools

import jax
import jax.numpy as jnp
from jax import lax
from jax.experimental import pallas as pl
from jax.experimental.pallas import tpu as pltpu
from jax.experimental.pallas import tpu_sc as plsc

N = 10000
E = 160000
D = 256
DH = 128

NC = 2
NS = 16
NW = NC * NS

K = 128
EPW = 5120
NCHUNK = EPW // K
E_PAD = NW * EPW
N_ACC = 10240
RPT = N_ACC // NS
TRASH = N_ACC - 8
DW = 16

_mesh = plsc.VectorSubcoreMesh(core_axis_name="c", subcore_axis_name="s")


def _deg_body(dstp_hbm, out_hbm, ones_v, didx_v, zbuf_v, acc):
    c = lax.axis_index("c")
    s = lax.axis_index("s")
    wid = c * NS + s

    @pl.loop(0, K)
    def _(i):
        ones_v[i, :] = jnp.ones((DW,), jnp.float32)

    @pl.loop(0, K)
    def _(i):
        zbuf_v[i, :] = jnp.zeros((DW,), jnp.float32)

    for j in range(RPT // K):
        pltpu.sync_copy(zbuf_v, acc.at[pl.ds(s * RPT + j * K, K)])
    plsc.subcore_barrier()

    @pl.loop(0, NCHUNK)
    def _(i):
        off = wid * EPW + i * K
        pltpu.sync_copy(dstp_hbm.at[pl.ds(off, K)], didx_v)
        pltpu.sync_copy(ones_v, acc.at[didx_v], add=True)

    plsc.subcore_barrier()
    pltpu.sync_copy(acc.at[pl.ds(s * RPT, RPT)],
                    out_hbm.at[c, pl.ds(s * RPT, RPT)])


def _deg_call(dstp):
    f = pl.kernel(
        _deg_body,
        out_type=jax.ShapeDtypeStruct((NC, N_ACC, DW), jnp.float32),
        mesh=_mesh,
        scratch_types=[
            pltpu.VMEM((K, DW), jnp.float32),
            pltpu.VMEM((K,), jnp.int32),
            pltpu.VMEM((K, DW), jnp.float32),
            pltpu.VMEM_SHARED((N_ACC, DW), jnp.float32),
        ],
    )
    return f(dstp)


def _mm_body(degp_ref, x_ref, w_ref, ylo_ref, yhi_ref, dinv_ref):
    dp = degp_ref[...]
    deg = dp[0, :, 0:1] + dp[1, :, 0:1] + 1.0
    dinv = lax.rsqrt(deg)
    xw = jnp.dot(x_ref[...], w_ref[...], preferred_element_type=jnp.float32)
    y = xw * dinv
    ylo_ref[...] = y[:, :DH]
    yhi_ref[...] = y[:, DH:]
    dinv_ref[...] = dinv


def _mm_call(degp, x, W):
    BN = 1000
    grid = (N // BN,)
    return pl.pallas_call(
        _mm_body,
        grid=grid,
        in_specs=[
            pl.BlockSpec((NC, BN, DW), lambda i: (0, i, 0)),
            pl.BlockSpec((BN, D), lambda i: (i, 0)),
            pl.BlockSpec((D, D), lambda i: (0, 0)),
        ],
        out_specs=[
            pl.BlockSpec((BN, DH), lambda i: (i, 0)),
            pl.BlockSpec((BN, DH), lambda i: (i, 0)),
            pl.BlockSpec((BN, 1), lambda i: (i, 0)),
        ],
        out_shape=[
            jax.ShapeDtypeStruct((N, DH), jnp.float32),
            jax.ShapeDtypeStruct((N, DH), jnp.float32),
            jax.ShapeDtypeStruct((N, 1), jnp.float32),
        ],
    )(degp, x, W)


def _agg_body(ylo_hbm, yhi_hbm, srcp_hbm, dstp_hbm, zeros_hbm, out_hbm,
              sidx_v, didx_v, rows_v, zbuf_v, acc, sem):
    c = lax.axis_index("c")
    s = lax.axis_index("s")
    wid = c * NS + s

    pltpu.sync_copy(zeros_hbm, zbuf_v)

    for h, y_hbm in enumerate((ylo_hbm, yhi_hbm)):
        for j in range(RPT // K):
            pltpu.sync_copy(zbuf_v, acc.at[pl.ds(s * RPT + j * K, K)])
        plsc.subcore_barrier()

        @pl.loop(0, NCHUNK)
        def _(i):
            off = wid * EPW + i * K
            pltpu.sync_copy(srcp_hbm.at[pl.ds(off, K)], sidx_v)
            pltpu.sync_copy(dstp_hbm.at[pl.ds(off, K)], didx_v)
            pltpu.async_copy(y_hbm.at[sidx_v], rows_v, sem).wait()
            pltpu.sync_copy(rows_v, acc.at[didx_v], add=True)

        plsc.subcore_barrier()
        pltpu.sync_copy(acc.at[pl.ds(s * RPT, RPT)],
                        out_hbm.at[h, c, pl.ds(s * RPT, RPT)])
        if h == 0:
            plsc.subcore_barrier()


def _agg_call(ylo, yhi, srcp, dstp, zeros):
    f = pl.kernel(
        _agg_body,
        out_type=jax.ShapeDtypeStruct((2, NC, N_ACC, DH), jnp.float32),
        mesh=_mesh,
        scratch_types=[
            pltpu.VMEM((K,), jnp.int32),
            pltpu.VMEM((K,), jnp.int32),
            pltpu.VMEM((K, DH), jnp.float32),
            pltpu.VMEM((K, DH), jnp.float32),
            pltpu.VMEM_SHARED((N_ACC, DH), jnp.float32),
            pltpu.SemaphoreType.DMA,
        ],
    )
    return f(ylo, yhi, srcp, dstp, zeros)


def _fin_body(s_ref, ylo_ref, yhi_ref, dinv_ref, b_ref, out_ref):
    sr = s_ref[...]
    lo = sr[0, 0] + sr[0, 1] + ylo_ref[...]
    hi = sr[1, 0] + sr[1, 1] + yhi_ref[...]
    t = jnp.concatenate([lo, hi], axis=1)
    out = t * dinv_ref[...] + b_ref[...]
    out_ref[...] = jnp.maximum(out, 0.0)


def _fin_call(sagg, ylo, yhi, dinv, b2):
    BN = 1000
    grid = (N // BN,)
    return pl.pallas_call(
        _fin_body,
        grid=grid,
        in_specs=[
            pl.BlockSpec((2, NC, BN, DH), lambda i: (0, 0, i, 0)),
            pl.BlockSpec((BN, DH), lambda i: (i, 0)),
            pl.BlockSpec((BN, DH), lambda i: (i, 0)),
            pl.BlockSpec((BN, 1), lambda i: (i, 0)),
            pl.BlockSpec((1, D), lambda i: (0, 0)),
        ],
        out_specs=pl.BlockSpec((BN, D), lambda i: (i, 0)),
        out_shape=jax.ShapeDtypeStruct((N, D), jnp.float32),
    )(sagg, ylo, yhi, dinv, b2)


@jax.jit
def kernel(x, edge_index, W, b):
    src = edge_index[0]
    dst = edge_index[1]
    pad = E_PAD - E
    srcp = jnp.concatenate([src, jnp.zeros((pad,), jnp.int32)])
    dstp = jnp.concatenate([dst, jnp.full((pad,), TRASH, jnp.int32)])
    zeros = jnp.zeros((K, DH), jnp.float32)

    degp = _deg_call(dstp)
    ylo, yhi, dinv = _mm_call(degp, x, W)
    sagg = _agg_call(ylo, yhi, srcp, dstp, zeros)
    b2 = b.reshape(1, D)
    return _fin_call(sagg, ylo, yhi, dinv, b2)


# speedup vs baseline: 7.4352x; 1.1826x over previous
import jax
import jax.numpy as jnp
from jax import lax
from jax.experimental import pallas as pl
from jax.experimental.pallas import tpu as pltpu
from jax.experimental.pallas import tpu_sc as plsc

N = 10000
E = 160000
D = 256
DH = 128

NC = 2
NS = 16
NW = NC * NS

K = 128
EPW = 5120
NCHUNK = EPW // K
E_PAD = NW * EPW
N_ACC = 10240
RPT = N_ACC // NS
TRASH = N_ACC - 8
DW = 16

_mesh = plsc.VectorSubcoreMesh(core_axis_name="c", subcore_axis_name="s")


def _deg_body(idx3_hbm, out_hbm, ones_v, ibuf_v, zbuf_v, acc, sem):
    c = lax.axis_index("c")
    s = lax.axis_index("s")
    wid = c * NS + s

    @pl.loop(0, K)
    def _(i):
        ones_v[i, :] = jnp.ones((DW,), jnp.float32)

    @pl.loop(0, K)
    def _(i):
        zbuf_v[i, :] = jnp.zeros((DW,), jnp.float32)

    pltpu.sync_copy(idx3_hbm.at[wid], ibuf_v)

    for j in range(RPT // K):
        pltpu.sync_copy(zbuf_v, acc.at[pl.ds(s * RPT + j * K, K)])
    plsc.subcore_barrier()

    descs = []
    for i in range(NCHUNK):
        descs.append(
            pltpu.async_copy(ones_v, acc.at[ibuf_v.at[2 * i + 1]], sem,
                             add=True))
    for d in descs:
        d.wait()

    plsc.subcore_barrier()
    pltpu.sync_copy(acc.at[pl.ds(s * RPT, RPT)],
                    out_hbm.at[c, pl.ds(s * RPT, RPT)])


def _deg_call(idx3):
    f = pl.kernel(
        _deg_body,
        out_type=jax.ShapeDtypeStruct((NC, N_ACC, DW), jnp.float32),
        mesh=_mesh,
        scratch_types=[
            pltpu.VMEM((K, DW), jnp.float32),
            pltpu.VMEM((2 * NCHUNK, K), jnp.int32),
            pltpu.VMEM((K, DW), jnp.float32),
            pltpu.VMEM_SHARED((N_ACC, DW), jnp.float32),
            pltpu.SemaphoreType.DMA,
        ],
    )
    return f(idx3)


def _mm_body(degp_ref, x_ref, w_ref, ylo_ref, yhi_ref, dinv_ref):
    dp = degp_ref[...]
    deg = dp[0, :, 0:1] + dp[1, :, 0:1] + 1.0
    dinv = lax.rsqrt(deg)
    xw = jnp.dot(x_ref[...], w_ref[...], preferred_element_type=jnp.float32)
    y = xw * dinv
    ylo_ref[...] = y[:, :DH]
    yhi_ref[...] = y[:, DH:]
    dinv_ref[...] = dinv


def _mm_call(degp, x, W):
    BN = 1000
    grid = (N // BN,)
    return pl.pallas_call(
        _mm_body,
        grid=grid,
        in_specs=[
            pl.BlockSpec((NC, BN, DW), lambda i: (0, i, 0)),
            pl.BlockSpec((BN, D), lambda i: (i, 0)),
            pl.BlockSpec((D, D), lambda i: (0, 0)),
        ],
        out_specs=[
            pl.BlockSpec((BN, DH), lambda i: (i, 0)),
            pl.BlockSpec((BN, DH), lambda i: (i, 0)),
            pl.BlockSpec((BN, 1), lambda i: (i, 0)),
        ],
        out_shape=[
            jax.ShapeDtypeStruct((N, DH), jnp.float32),
            jax.ShapeDtypeStruct((N, DH), jnp.float32),
            jax.ShapeDtypeStruct((N, 1), jnp.float32),
        ],
    )(degp, x, W)


def _agg_body(ylo_hbm, yhi_hbm, idx3_hbm, zeros_hbm, out_hbm,
              ibuf_v, rows0_v, rows1_v, acc, gsem0, gsem1):
    c = lax.axis_index("c")
    s = lax.axis_index("s")
    wid = c * NS + s

    pltpu.sync_copy(idx3_hbm.at[wid], ibuf_v)

    for h, y_hbm in enumerate((ylo_hbm, yhi_hbm)):
        pltpu.sync_copy(zeros_hbm, rows0_v)
        for j in range(RPT // K):
            pltpu.sync_copy(rows0_v, acc.at[pl.ds(s * RPT + j * K, K)])
        plsc.subcore_barrier()

        pltpu.async_copy(y_hbm.at[ibuf_v.at[0]], rows0_v, gsem0)

        @pl.loop(0, NCHUNK // 2)
        def _(j):
            i0 = 2 * j
            pltpu.make_async_copy(
                y_hbm.at[ibuf_v.at[2 * i0]], rows0_v, gsem0).wait()
            pltpu.async_copy(
                y_hbm.at[ibuf_v.at[2 * (i0 + 1)]], rows1_v, gsem1)
            pltpu.sync_copy(rows0_v, acc.at[ibuf_v.at[2 * i0 + 1]], add=True)
            pltpu.make_async_copy(
                y_hbm.at[ibuf_v.at[2 * (i0 + 1)]], rows1_v, gsem1).wait()

            @pl.when(j < NCHUNK // 2 - 1)
            def _():
                pltpu.async_copy(
                    y_hbm.at[ibuf_v.at[2 * (i0 + 2)]], rows0_v, gsem0)

            pltpu.sync_copy(rows1_v, acc.at[ibuf_v.at[2 * i0 + 3]], add=True)

        plsc.subcore_barrier()
        pltpu.sync_copy(acc.at[pl.ds(s * RPT, RPT)],
                        out_hbm.at[h, c, pl.ds(s * RPT, RPT)])
        if h == 0:
            plsc.subcore_barrier()


def _agg_call(ylo, yhi, idx3, zeros):
    f = pl.kernel(
        _agg_body,
        out_type=jax.ShapeDtypeStruct((2, NC, N_ACC, DH), jnp.float32),
        mesh=_mesh,
        scratch_types=[
            pltpu.VMEM((2 * NCHUNK, K), jnp.int32),
            pltpu.VMEM((K, DH), jnp.float32),
            pltpu.VMEM((K, DH), jnp.float32),
            pltpu.VMEM_SHARED((N_ACC, DH), jnp.float32),
            pltpu.SemaphoreType.DMA,
            pltpu.SemaphoreType.DMA,
        ],
    )
    return f(ylo, yhi, idx3, zeros)


def _fin_body(s_ref, ylo_ref, yhi_ref, dinv_ref, b_ref, out_ref):
    sr = s_ref[...]
    lo = sr[0, 0] + sr[0, 1] + ylo_ref[...]
    hi = sr[1, 0] + sr[1, 1] + yhi_ref[...]
    t = jnp.concatenate([lo, hi], axis=1)
    out = t * dinv_ref[...] + b_ref[...]
    out_ref[...] = jnp.maximum(out, 0.0)


def _fin_call(sagg, ylo, yhi, dinv, b2):
    BN = 1000
    grid = (N // BN,)
    return pl.pallas_call(
        _fin_body,
        grid=grid,
        in_specs=[
            pl.BlockSpec((2, NC, BN, DH), lambda i: (0, 0, i, 0)),
            pl.BlockSpec((BN, DH), lambda i: (i, 0)),
            pl.BlockSpec((BN, DH), lambda i: (i, 0)),
            pl.BlockSpec((BN, 1), lambda i: (i, 0)),
            pl.BlockSpec((1, D), lambda i: (0, 0)),
        ],
        out_specs=pl.BlockSpec((BN, D), lambda i: (i, 0)),
        out_shape=jax.ShapeDtypeStruct((N, D), jnp.float32),
    )(sagg, ylo, yhi, dinv, b2)


@jax.jit
def kernel(x, edge_index, W, b):
    src = edge_index[0]
    dst = edge_index[1]
    pad = E_PAD - E
    srcp = jnp.concatenate([src, jnp.zeros((pad,), jnp.int32)])
    dstp = jnp.concatenate([dst, jnp.full((pad,), TRASH, jnp.int32)])
    idx3 = jnp.stack(
        [srcp.reshape(NW, NCHUNK, K), dstp.reshape(NW, NCHUNK, K)], axis=2
    ).reshape(NW, 2 * NCHUNK, K)
    zeros = jnp.zeros((K, DH), jnp.float32)

    degp = _deg_call(idx3)
    ylo, yhi, dinv = _mm_call(degp, x, W)
    sagg = _agg_call(ylo, yhi, idx3, zeros)
    b2 = b.reshape(1, D)
    return _fin_call(sagg, ylo, yhi, dinv, b2)


# speedup vs baseline: 7.4469x; 1.0016x over previous
import jax
import jax.numpy as jnp
from jax import lax
from jax.experimental import pallas as pl
from jax.experimental.pallas import tpu as pltpu
from jax.experimental.pallas import tpu_sc as plsc

N = 10000
E = 160000
D = 256
DH = 128

NC = 2
NS = 16
NW = NC * NS

K = 128
EPW = 5120
NCHUNK = EPW // K
E_PAD = NW * EPW
N_ACC = 10240
RPT = N_ACC // NS
TRASH = N_ACC - 8
DW = 16
NBUF = 2

_mesh = plsc.VectorSubcoreMesh(core_axis_name="c", subcore_axis_name="s")


def _deg_body(idx3_hbm, out_hbm, ones_v, ibuf_v, zbuf_v, acc, sem):
    c = lax.axis_index("c")
    s = lax.axis_index("s")
    wid = c * NS + s

    @pl.loop(0, K)
    def _(i):
        ones_v[i, :] = jnp.ones((DW,), jnp.float32)

    @pl.loop(0, K)
    def _(i):
        zbuf_v[i, :] = jnp.zeros((DW,), jnp.float32)

    pltpu.sync_copy(idx3_hbm.at[wid], ibuf_v)

    for j in range(RPT // K):
        pltpu.sync_copy(zbuf_v, acc.at[pl.ds(s * RPT + j * K, K)])
    plsc.subcore_barrier()

    descs = []
    for i in range(NCHUNK):
        descs.append(
            pltpu.async_copy(ones_v, acc.at[ibuf_v.at[2 * i + 1]], sem,
                             add=True))
    for d in descs:
        d.wait()

    plsc.subcore_barrier()
    pltpu.sync_copy(acc.at[pl.ds(s * RPT, RPT)],
                    out_hbm.at[c, pl.ds(s * RPT, RPT)])


def _deg_call(idx3):
    f = pl.kernel(
        _deg_body,
        out_type=jax.ShapeDtypeStruct((NC, N_ACC, DW), jnp.float32),
        mesh=_mesh,
        scratch_types=[
            pltpu.VMEM((K, DW), jnp.float32),
            pltpu.VMEM((2 * NCHUNK, K), jnp.int32),
            pltpu.VMEM((K, DW), jnp.float32),
            pltpu.VMEM_SHARED((N_ACC, DW), jnp.float32),
            pltpu.SemaphoreType.DMA,
        ],
    )
    return f(idx3)


def _mm_body(degp_ref, x_ref, w_ref, ylo_ref, yhi_ref, dinv_ref):
    dp = degp_ref[...]
    deg = dp[0, :, 0:1] + dp[1, :, 0:1] + 1.0
    dinv = lax.rsqrt(deg)
    xw = jnp.dot(x_ref[...], w_ref[...], preferred_element_type=jnp.float32)
    y = xw * dinv
    ylo_ref[...] = y[:, :DH]
    yhi_ref[...] = y[:, DH:]
    dinv_ref[...] = dinv


def _mm_call(degp, x, W):
    BN = 1000
    grid = (N // BN,)
    return pl.pallas_call(
        _mm_body,
        grid=grid,
        in_specs=[
            pl.BlockSpec((NC, BN, DW), lambda i: (0, i, 0)),
            pl.BlockSpec((BN, D), lambda i: (i, 0)),
            pl.BlockSpec((D, D), lambda i: (0, 0)),
        ],
        out_specs=[
            pl.BlockSpec((BN, DH), lambda i: (i, 0)),
            pl.BlockSpec((BN, DH), lambda i: (i, 0)),
            pl.BlockSpec((BN, 1), lambda i: (i, 0)),
        ],
        out_shape=[
            jax.ShapeDtypeStruct((N, DH), jnp.float32),
            jax.ShapeDtypeStruct((N, DH), jnp.float32),
            jax.ShapeDtypeStruct((N, 1), jnp.float32),
        ],
    )(degp, x, W)


def _agg_body(ylo_hbm, yhi_hbm, idx3_hbm, zeros_hbm, out_hbm,
              ibuf_v, r0, r1,
              acc, g0, g1, s0, s1):
    c = lax.axis_index("c")
    s = lax.axis_index("s")
    wid = c * NS + s
    rows = (r0, r1)
    gsem = (g0, g1)
    ssem = (s0, s1)

    pltpu.sync_copy(idx3_hbm.at[wid], ibuf_v)

    for h, y_hbm in enumerate((ylo_hbm, yhi_hbm)):
        pltpu.sync_copy(zeros_hbm, rows[0])
        for j in range(RPT // K):
            pltpu.sync_copy(rows[0], acc.at[pl.ds(s * RPT + j * K, K)])
        plsc.subcore_barrier()

        def wait_g(i, b):
            pltpu.make_async_copy(
                y_hbm.at[ibuf_v.at[2 * i]], rows[b], gsem[b]).wait()

        def fire_s(i, b):
            pltpu.async_copy(
                rows[b], acc.at[ibuf_v.at[2 * i + 1]], ssem[b], add=True)

        def wait_s(i, b):
            pltpu.make_async_copy(
                rows[b], acc.at[ibuf_v.at[2 * i + 1]], ssem[b]).wait()

        def fire_g(i, b):
            pltpu.async_copy(y_hbm.at[ibuf_v.at[2 * i]], rows[b], gsem[b])

        fire_g(0, 0)
        wait_g(0, 0)
        fire_s(0, 0)
        fire_g(1, 1)

        @pl.loop(0, NCHUNK // 2 - 1)
        def _(g):
            i0 = 2 * g + 1
            wait_g(i0, 1)
            fire_s(i0, 1)
            wait_s(i0 - 1, 0)
            fire_g(i0 + 1, 0)
            i1 = i0 + 1
            wait_g(i1, 0)
            fire_s(i1, 0)
            wait_s(i1 - 1, 1)
            fire_g(i1 + 1, 1)

        wait_g(NCHUNK - 1, 1)
        fire_s(NCHUNK - 1, 1)
        wait_s(NCHUNK - 2, 0)
        wait_s(NCHUNK - 1, 1)

        plsc.subcore_barrier()
        pltpu.sync_copy(acc.at[pl.ds(s * RPT, RPT)],
                        out_hbm.at[h, c, pl.ds(s * RPT, RPT)])
        if h == 0:
            plsc.subcore_barrier()


def _agg_call(ylo, yhi, idx3, zeros):
    f = pl.kernel(
        _agg_body,
        out_type=jax.ShapeDtypeStruct((2, NC, N_ACC, DH), jnp.float32),
        mesh=_mesh,
        scratch_types=(
            [pltpu.VMEM((2 * NCHUNK, K), jnp.int32)] +
            [pltpu.VMEM((K, DH), jnp.float32)] * NBUF +
            [pltpu.VMEM_SHARED((N_ACC, DH), jnp.float32)] +
            [pltpu.SemaphoreType.DMA] * (2 * NBUF)
        ),
    )
    return f(ylo, yhi, idx3, zeros)


def _fin_body(s_ref, ylo_ref, yhi_ref, dinv_ref, b_ref, out_ref):
    sr = s_ref[...]
    lo = sr[0, 0] + sr[0, 1] + ylo_ref[...]
    hi = sr[1, 0] + sr[1, 1] + yhi_ref[...]
    t = jnp.concatenate([lo, hi], axis=1)
    out = t * dinv_ref[...] + b_ref[...]
    out_ref[...] = jnp.maximum(out, 0.0)


def _fin_call(sagg, ylo, yhi, dinv, b2):
    BN = 1000
    grid = (N // BN,)
    return pl.pallas_call(
        _fin_body,
        grid=grid,
        in_specs=[
            pl.BlockSpec((2, NC, BN, DH), lambda i: (0, 0, i, 0)),
            pl.BlockSpec((BN, DH), lambda i: (i, 0)),
            pl.BlockSpec((BN, DH), lambda i: (i, 0)),
            pl.BlockSpec((BN, 1), lambda i: (i, 0)),
            pl.BlockSpec((1, D), lambda i: (0, 0)),
        ],
        out_specs=pl.BlockSpec((BN, D), lambda i: (i, 0)),
        out_shape=jax.ShapeDtypeStruct((N, D), jnp.float32),
    )(sagg, ylo, yhi, dinv, b2)


@jax.jit
def kernel(x, edge_index, W, b):
    src = edge_index[0]
    dst = edge_index[1]
    pad = E_PAD - E
    srcp = jnp.concatenate([src, jnp.zeros((pad,), jnp.int32)])
    dstp = jnp.concatenate([dst, jnp.full((pad,), TRASH, jnp.int32)])
    idx3 = jnp.stack(
        [srcp.reshape(NW, NCHUNK, K), dstp.reshape(NW, NCHUNK, K)], axis=2
    ).reshape(NW, 2 * NCHUNK, K)
    zeros = jnp.zeros((K, DH), jnp.float32)

    degp = _deg_call(idx3)
    ylo, yhi, dinv = _mm_call(degp, x, W)
    sagg = _agg_call(ylo, yhi, idx3, zeros)
    b2 = b.reshape(1, D)
    return _fin_call(sagg, ylo, yhi, dinv, b2)
